# baseline (speedup 1.0000x reference)
import functools
import jax
import jax.numpy as jnp
from jax import lax
from jax.experimental import pallas as pl
from jax.experimental.pallas import tpu as pltpu

N = 10000
E = 160000
D = 256
ED = 16

NBLK = 10
NROWS = N // NBLK
EROWS = 128
EBLK = (E // 128 + EROWS - 1) // EROWS


def _stage1_body(x_ref, wn_ref, wa_ref, z_ref, s_ref):
    z = jnp.dot(x_ref[...], wn_ref[...], preferred_element_type=jnp.float32)
    z_ref[...] = z
    s_ref[...] = jnp.dot(z, wa_ref[...], preferred_element_type=jnp.float32)


def _stage1(x, Wn, wa_pair):
    return pl.pallas_call(
        _stage1_body,
        grid=(NBLK,),
        in_specs=[
            pl.BlockSpec((NROWS, D), lambda i: (i, 0)),
            pl.BlockSpec((D, D), lambda i: (0, 0)),
            pl.BlockSpec((D, 128), lambda i: (0, 0)),
        ],
        out_specs=[
            pl.BlockSpec((NROWS, D), lambda i: (i, 0)),
            pl.BlockSpec((NROWS, 128), lambda i: (i, 0)),
        ],
        out_shape=[
            jax.ShapeDtypeStruct((N, D), jnp.float32),
            jax.ShapeDtypeStruct((N, 128), jnp.float32),
        ],
    )(x, Wn, wa_pair)


def _stage_c_body(ef_ref, we_ref, wam_ref, c_ref):
    w_ec = jnp.dot(we_ref[...], wam_ref[...], preferred_element_type=jnp.float32)
    ef = ef_ref[...]
    c_ref[...] = jnp.sum(ef * w_ec[:, 0][None, None, :], axis=-1)


def _stage_c(ef3, We, wa_mid):
    return pl.pallas_call(
        _stage_c_body,
        grid=(EBLK,),
        in_specs=[
            pl.BlockSpec((EROWS, 128, ED), lambda i: (i, 0, 0)),
            pl.BlockSpec((ED, ED), lambda i: (0, 0)),
            pl.BlockSpec((ED, 1), lambda i: (0, 0)),
        ],
        out_specs=pl.BlockSpec((EROWS, 128), lambda i: (i, 0)),
        out_shape=jax.ShapeDtypeStruct((E // 128, 128), jnp.float32),
    )(ef3, We, wa_mid)


def _stage3_body(p_ref, r_ref, z_ref, den_ref, w2a_ref, we_ref, w2b_ref, o_ref):
    w2c = jnp.dot(we_ref[...], w2b_ref[...], preferred_element_type=jnp.float32)
    h = jnp.dot(p_ref[...], w2a_ref[...], preferred_element_type=jnp.float32)
    h = h + jnp.dot(r_ref[...], w2c, preferred_element_type=jnp.float32)
    o_ref[...] = jnp.where(den_ref[...] > 0.0, h, z_ref[...])


def _stage3(P, R, z, denom, W2a, We, W2b):
    return pl.pallas_call(
        _stage3_body,
        grid=(NBLK,),
        in_specs=[
            pl.BlockSpec((NROWS, D), lambda i: (i, 0)),
            pl.BlockSpec((NROWS, ED), lambda i: (i, 0)),
            pl.BlockSpec((NROWS, D), lambda i: (i, 0)),
            pl.BlockSpec((NROWS, 1), lambda i: (i, 0)),
            pl.BlockSpec((D, D), lambda i: (0, 0)),
            pl.BlockSpec((ED, ED), lambda i: (0, 0)),
            pl.BlockSpec((ED, D), lambda i: (0, 0)),
        ],
        out_specs=pl.BlockSpec((NROWS, D), lambda i: (i, 0)),
        out_shape=jax.ShapeDtypeStruct((N, D), jnp.float32),
    )(P, R, z, denom, W2a, We, W2b)


def kernel(node_features, edges_features, edge_index, Wn, We, Wa, W2):
    src = edge_index[0]
    dst = edge_index[1]

    wa_pair = jnp.zeros((D, 128), jnp.float32)
    wa_pair = wa_pair.at[:, 0].set(Wa[:D, 0]).at[:, 1].set(Wa[D + ED:, 0])
    z, s_pair = _stage1(node_features, Wn, wa_pair)
    s1 = s_pair[:, 0]
    s2 = s_pair[:, 1]

    ef3 = edges_features.reshape(E // 128, 128, ED)
    c = _stage_c(ef3, We, Wa[D:D + ED]).reshape(E)

    a = s1[src] + c + s2[dst]
    e = jnp.where(a > 0, a, 0.01 * a)
    gmax = jnp.max(e)
    ex = jnp.exp(e - gmax)
    denom = jax.ops.segment_sum(ex, dst, num_segments=N)
    alpha = ex / denom[dst]
    P = jax.ops.segment_sum(alpha[:, None] * z[src], dst, num_segments=N)
    R = jax.ops.segment_sum(alpha[:, None] * edges_features, dst, num_segments=N)

    return _stage3(P, R, z, denom[:, None], W2[:D], We, W2[D:])


# speedup vs baseline: 5.1563x; 5.1563x over previous
import functools
import jax
import jax.numpy as jnp
from jax import lax
from jax.experimental import pallas as pl
from jax.experimental.pallas import tpu as pltpu
from jax.experimental.pallas import tpu_sc as plsc

N = 10000
E = 160000
D = 256
ED = 16

NBLK = 10
NROWS = N // NBLK
EROWS = 128
EBLK = (E // 128 + EROWS - 1) // EROWS

ET = E // 16
C = 80
NCH = ET // C
NSL = 624
VSTEPS = ET // 16
H = 64


def _stage1_body(x_ref, wn_ref, wa_ref, z0_ref, z1_ref, z2_ref, z3_ref, s_ref):
    z = jnp.dot(x_ref[...], wn_ref[...], preferred_element_type=jnp.float32)
    z0_ref[...] = z[:, 0:64]
    z1_ref[...] = z[:, 64:128]
    z2_ref[...] = z[:, 128:192]
    z3_ref[...] = z[:, 192:256]
    s_ref[...] = jnp.dot(z, wa_ref[...], preferred_element_type=jnp.float32)


def _stage1(x, Wn, wa_pair):
    zspec = pl.BlockSpec((NROWS, H), lambda i: (i, 0))
    zshape = jax.ShapeDtypeStruct((N, H), jnp.float32)
    return pl.pallas_call(
        _stage1_body,
        grid=(NBLK,),
        in_specs=[
            pl.BlockSpec((NROWS, D), lambda i: (i, 0)),
            pl.BlockSpec((D, D), lambda i: (0, 0)),
            pl.BlockSpec((D, 128), lambda i: (0, 0)),
        ],
        out_specs=[zspec, zspec, zspec, zspec,
                   pl.BlockSpec((NROWS, 128), lambda i: (i, 0))],
        out_shape=[zshape, zshape, zshape, zshape,
                   jax.ShapeDtypeStruct((N, 128), jnp.float32)],
    )(x, Wn, wa_pair)


def _stage_c_body(ef_ref, we_ref, wam_ref, c_ref):
    w_ec = jnp.dot(we_ref[...], wam_ref[...], preferred_element_type=jnp.float32)
    ef = ef_ref[...]
    c_ref[...] = jnp.sum(ef * w_ec[:, 0][None, None, :], axis=-1)


def _stage_c(ef3, We, wa_mid):
    return pl.pallas_call(
        _stage_c_body,
        grid=(EBLK,),
        in_specs=[
            pl.BlockSpec((EROWS, 128, ED), lambda i: (i, 0, 0)),
            pl.BlockSpec((ED, ED), lambda i: (0, 0)),
            pl.BlockSpec((ED, 1), lambda i: (0, 0)),
        ],
        out_specs=pl.BlockSpec((EROWS, 128), lambda i: (i, 0)),
        out_shape=jax.ShapeDtypeStruct((E // 128, 128), jnp.float32),
    )(ef3, We, wa_mid)


def _stage3_body(p0_ref, p1_ref, p2_ref, p3_ref, r_ref,
                 z0_ref, z1_ref, z2_ref, z3_ref, den_ref,
                 w2a_ref, we_ref, w2b_ref, o_ref):
    w2c = jnp.dot(we_ref[...], w2b_ref[...], preferred_element_type=jnp.float32)
    p = jnp.concatenate(
        [p0_ref[...], p1_ref[...], p2_ref[...], p3_ref[...]], axis=1)
    h = jnp.dot(p, w2a_ref[...], preferred_element_type=jnp.float32)
    h = h + jnp.dot(r_ref[...], w2c, preferred_element_type=jnp.float32)
    z = jnp.concatenate(
        [z0_ref[...], z1_ref[...], z2_ref[...], z3_ref[...]], axis=1)
    o_ref[...] = jnp.where(den_ref[...] > 0.0, h, z)


def _stage3(ps, R, zs, denom, W2a, We, W2b):
    zspec = pl.BlockSpec((NROWS, H), lambda i: (i, 0))
    return pl.pallas_call(
        _stage3_body,
        grid=(NBLK,),
        in_specs=[
            zspec, zspec, zspec, zspec,
            pl.BlockSpec((NROWS, ED), lambda i: (i, 0)),
            zspec, zspec, zspec, zspec,
            pl.BlockSpec((NROWS, 1), lambda i: (i, 0)),
            pl.BlockSpec((D, D), lambda i: (0, 0)),
            pl.BlockSpec((ED, ED), lambda i: (0, 0)),
            pl.BlockSpec((ED, D), lambda i: (0, 0)),
        ],
        out_specs=pl.BlockSpec((NROWS, D), lambda i: (i, 0)),
        out_shape=jax.ShapeDtypeStruct((N, D), jnp.float32),
    )(*ps, R, *zs, denom, W2a, We, W2b)


def _f16(v):
    return jnp.full((16,), v, jnp.float32)


def _sc_edge_body(src_hbm, dst_hbm, c_hbm, s1_hbm, s2_hbm,
                  zq0_hbm, zq1_hbm, zq2_hbm, zq3_hbm, ef_hbm,
                  p0_out, p1_out, p2_out, p3_out, r_out, den_out,
                  s1_v, s2_v, src_v, dst_v, ac_v, den_v,
                  buf, zbuf, ef_buf, zrow, vec16, m16,
                  p_sp, r_sp, den_sp, gmax_sp, sem):
    t = lax.axis_index("s")
    cid = lax.axis_index("c")
    base = t * ET

    pltpu.sync_copy(s1_hbm, s1_v)
    pltpu.sync_copy(s2_hbm, s2_v)
    pltpu.sync_copy(src_hbm.at[t], src_v)
    pltpu.sync_copy(dst_hbm.at[t], dst_v)
    pltpu.sync_copy(c_hbm.at[t], ac_v)

    zero16 = _f16(0.0)

    def _zb(r, _):
        for q in range(H // 16):
            zbuf[r, pl.ds(q * 16, 16)] = zero16
        return 0
    lax.fori_loop(0, 104, _zb, 0)

    def _ze(r, _):
        ef_buf[r, pl.ds(0, 16)] = zero16
        return 0
    lax.fori_loop(0, 128, _ze, 0)

    def _zr(i, _):
        zrow[pl.ds(i * 16, 16)] = zero16
        return 0
    lax.fori_loop(0, 62, _zr, 0)
    zrow[pl.ds(984, 16)] = zero16

    def _zero_p_sp():
        for k in range(6):
            pltpu.sync_copy(zbuf.at[pl.ds(0, 104)],
                            p_sp.at[pl.ds(t * NSL + k * 104, 104)])

        @pl.when(t == 15)
        def _():
            pltpu.sync_copy(zbuf.at[pl.ds(0, 16)], p_sp.at[pl.ds(9984, 16)])

    _zero_p_sp()
    for k in range(6):
        pltpu.sync_copy(ef_buf.at[pl.ds(0, 104)],
                        r_sp.at[pl.ds(t * NSL + k * 104, 104)])

    @pl.when(t == 15)
    def _():
        pltpu.sync_copy(ef_buf.at[pl.ds(0, 16)], r_sp.at[pl.ds(9984, 16)])

    @pl.when(t < 10)
    def _():
        pltpu.sync_copy(zrow, den_sp.at[pl.ds(t * 1000, 1000)])

    def _pass_a(j, m):
        for k in range(C // 16):
            sl = pl.ds(k * 16, 16)
            sv = src_v[j, sl]
            dv = dst_v[j, sl]
            g1 = plsc.load_gather(s1_v, [sv])
            g2 = plsc.load_gather(s2_v, [dv])
            a = g1 + ac_v[j, sl] + g2
            e = jnp.where(a > 0.0, a, 0.01 * a)
            ac_v[j, sl] = e
            m = jnp.maximum(m, e)
        return m
    m = lax.fori_loop(0, NCH, _pass_a, _f16(-1e30))

    vec16[pl.ds(0, 16)] = m
    pltpu.sync_copy(vec16, gmax_sp.at[t])
    plsc.subcore_barrier()
    pltpu.sync_copy(gmax_sp, m16)

    def _mred(j, mm):
        return jnp.maximum(mm, m16[j, pl.ds(0, 16)])
    m = lax.fori_loop(0, 16, _mred, _f16(-1e30))
    gv = _f16(jnp.max(m))

    def _pass_b(j, _):
        for k in range(C // 16):
            sl = pl.ds(k * 16, 16)
            ac_v[j, sl] = jnp.exp(ac_v[j, sl] - gv)
        pltpu.sync_copy(ac_v.at[j], den_sp.at[dst_v.at[j]], add=True)
        return 0
    lax.fori_loop(0, NCH, _pass_b, 0)
    plsc.subcore_barrier()

    pltpu.sync_copy(den_sp, den_v)

    def _pass_c(j, _):
        for k in range(C // 16):
            sl = pl.ds(k * 16, 16)
            dv = dst_v[j, sl]
            dsum = plsc.load_gather(den_v, [dv])
            ac_v[j, sl] = ac_v[j, sl] / dsum
        return 0
    lax.fori_loop(0, NCH, _pass_c, 0)

    def _scale_rows(j, ref, nq):
        def _r(r, _):
            jv = jnp.full((16,), j, jnp.int32)
            rv = jnp.full((16,), r, jnp.int32)
            av = plsc.load_gather(ac_v, [jv, rv])
            for q in range(nq):
                sl = pl.ds(q * 16, 16)
                ref[r, sl] = ref[r, sl] * av
            return 0
        lax.fori_loop(0, C, _r, 0)

    def _heavy(tbl, with_r):
        def _chunk(j, _):
            pltpu.async_copy(tbl.at[src_v.at[j]], buf.at[pl.ds(0, C)],
                             sem).wait()
            _scale_rows(j, buf, H // 16)
            pltpu.sync_copy(buf.at[pl.ds(0, C)],
                            p_sp.at[dst_v.at[j]], add=True)
            if with_r:
                pltpu.sync_copy(ef_hbm.at[pl.ds(base + j * C, C)],
                                ef_buf.at[pl.ds(0, C)])
                _scale_rows(j, ef_buf, 1)
                pltpu.sync_copy(ef_buf.at[pl.ds(0, C)],
                                r_sp.at[dst_v.at[j]], add=True)
            return 0
        lax.fori_loop(0, NCH, _chunk, 0)

    def _write_p(pout):
        rows = pl.ds(t * NSL, NSL)
        pltpu.sync_copy(p_sp.at[rows], pout.at[rows])

        @pl.when(t == 15)
        def _():
            tail = pl.ds(9984, 16)
            pltpu.sync_copy(p_sp.at[tail], pout.at[tail])

    @pl.when(cid == 0)
    def _():
        _heavy(zq0_hbm, True)

    @pl.when(cid == 1)
    def _():
        _heavy(zq2_hbm, False)

    plsc.subcore_barrier()

    rows = pl.ds(t * NSL, NSL)
    tail = pl.ds(9984, 16)

    @pl.when(cid == 0)
    def _():
        _write_p(p0_out)
        pltpu.sync_copy(r_sp.at[rows], r_out.at[rows])

    @pl.when(jnp.logical_and(cid == 0, t == 15))
    def _():
        pltpu.sync_copy(r_sp.at[tail], r_out.at[tail])

    @pl.when(jnp.logical_and(cid == 0, t < 10))
    def _():
        sl1k = pl.ds(t * 1000, 1000)
        pltpu.sync_copy(den_v.at[sl1k], den_out.at[sl1k])

    @pl.when(cid == 1)
    def _():
        _write_p(p2_out)

    _zero_p_sp()
    plsc.subcore_barrier()

    @pl.when(cid == 0)
    def _():
        _heavy(zq1_hbm, False)

    @pl.when(cid == 1)
    def _():
        _heavy(zq3_hbm, False)

    plsc.subcore_barrier()

    @pl.when(cid == 0)
    def _():
        _write_p(p1_out)

    @pl.when(cid == 1)
    def _():
        _write_p(p3_out)


@functools.partial(
    pl.kernel,
    mesh=plsc.VectorSubcoreMesh(core_axis_name="c", subcore_axis_name="s"),
    compiler_params=pltpu.CompilerParams(
        needs_layout_passes=False, use_tc_tiling_on_sc=False),
    out_type=[
        jax.ShapeDtypeStruct((N, H), jnp.float32),
        jax.ShapeDtypeStruct((N, H), jnp.float32),
        jax.ShapeDtypeStruct((N, H), jnp.float32),
        jax.ShapeDtypeStruct((N, H), jnp.float32),
        jax.ShapeDtypeStruct((N, ED), jnp.float32),
        jax.ShapeDtypeStruct((N,), jnp.float32),
    ],
    scratch_types=[
        pltpu.VMEM((N,), jnp.float32),
        pltpu.VMEM((N,), jnp.float32),
        pltpu.VMEM((NCH, C), jnp.int32),
        pltpu.VMEM((NCH, C), jnp.int32),
        pltpu.VMEM((NCH, C), jnp.float32),
        pltpu.VMEM((N,), jnp.float32),
        pltpu.VMEM((128, H), jnp.float32),
        pltpu.VMEM((104, H), jnp.float32),
        pltpu.VMEM((128, ED), jnp.float32),
        pltpu.VMEM((1000,), jnp.float32),
        pltpu.VMEM((16,), jnp.float32),
        pltpu.VMEM((16, 16), jnp.float32),
        pltpu.VMEM_SHARED((N, H), jnp.float32),
        pltpu.VMEM_SHARED((N, ED), jnp.float32),
        pltpu.VMEM_SHARED((N,), jnp.float32),
        pltpu.VMEM_SHARED((16, 16), jnp.float32),
        pltpu.SemaphoreType.DMA,
    ],
)
def _sc_edge(src, dst, c, s1, s2, zq0, zq1, zq2, zq3, ef,
             p0, p1, p2, p3, r, den, *scratch):
    _sc_edge_body(src, dst, c, s1, s2, zq0, zq1, zq2, zq3, ef,
                  p0, p1, p2, p3, r, den, *scratch)


def kernel(node_features, edges_features, edge_index, Wn, We, Wa, W2):
    src = edge_index[0]
    dst = edge_index[1]

    wa_pair = jnp.zeros((D, 128), jnp.float32)
    wa_pair = wa_pair.at[:, 0].set(Wa[:D, 0]).at[:, 1].set(Wa[D + ED:, 0])
    zq0, zq1, zq2, zq3, s_pair = _stage1(node_features, Wn, wa_pair)
    s1 = s_pair[:, 0]
    s2 = s_pair[:, 1]

    ef3 = edges_features.reshape(E // 128, 128, ED)
    c = _stage_c(ef3, We, Wa[D:D + ED]).reshape(E)

    src3 = src.reshape(16, NCH, C)
    dst3 = dst.reshape(16, NCH, C)
    c3 = c.reshape(16, NCH, C)
    p0, p1, p2, p3, R, denom = _sc_edge(src3, dst3, c3, s1, s2,
                                        zq0, zq1, zq2, zq3, edges_features)

    return _stage3((p0, p1, p2, p3), R, (zq0, zq1, zq2, zq3), denom[:, None],
                   W2[:D], We, W2[D:])


# speedup vs baseline: 8.2837x; 1.6065x over previous
import functools
import jax
import jax.numpy as jnp
from jax import lax
from jax.experimental import pallas as pl
from jax.experimental.pallas import tpu as pltpu
from jax.experimental.pallas import tpu_sc as plsc

N = 10000
E = 160000
D = 256
ED = 16

NBLK = 10
NROWS = N // NBLK
EROWS = 128
EBLK = (E // 128 + EROWS - 1) // EROWS

ET = E // 16
C = 80
NCH = ET // C
NSL = 624
VSTEPS = ET // 16
H = 64


def _stage1_body(x_ref, wn_ref, wa_ref, z0_ref, z1_ref, z2_ref, z3_ref, s_ref):
    z = jnp.dot(x_ref[...], wn_ref[...], preferred_element_type=jnp.float32)
    z0_ref[...] = z[:, 0:64]
    z1_ref[...] = z[:, 64:128]
    z2_ref[...] = z[:, 128:192]
    z3_ref[...] = z[:, 192:256]
    s_ref[...] = jnp.dot(z, wa_ref[...], preferred_element_type=jnp.float32)


def _stage1(x, Wn, wa_pair):
    zspec = pl.BlockSpec((NROWS, H), lambda i: (i, 0))
    zshape = jax.ShapeDtypeStruct((N, H), jnp.float32)
    return pl.pallas_call(
        _stage1_body,
        grid=(NBLK,),
        in_specs=[
            pl.BlockSpec((NROWS, D), lambda i: (i, 0)),
            pl.BlockSpec((D, D), lambda i: (0, 0)),
            pl.BlockSpec((D, 128), lambda i: (0, 0)),
        ],
        out_specs=[zspec, zspec, zspec, zspec,
                   pl.BlockSpec((NROWS, 128), lambda i: (i, 0))],
        out_shape=[zshape, zshape, zshape, zshape,
                   jax.ShapeDtypeStruct((N, 128), jnp.float32)],
    )(x, Wn, wa_pair)


def _stage_c_body(ef_ref, we_ref, wam_ref, c_ref):
    w_ec = jnp.dot(we_ref[...], wam_ref[...], preferred_element_type=jnp.float32)
    ef = ef_ref[...]
    c_ref[...] = jnp.sum(ef * w_ec[:, 0][None, None, :], axis=-1)


def _stage_c(ef3, We, wa_mid):
    return pl.pallas_call(
        _stage_c_body,
        grid=(EBLK,),
        in_specs=[
            pl.BlockSpec((EROWS, 128, ED), lambda i: (i, 0, 0)),
            pl.BlockSpec((ED, ED), lambda i: (0, 0)),
            pl.BlockSpec((ED, 1), lambda i: (0, 0)),
        ],
        out_specs=pl.BlockSpec((EROWS, 128), lambda i: (i, 0)),
        out_shape=jax.ShapeDtypeStruct((E // 128, 128), jnp.float32),
    )(ef3, We, wa_mid)


def _stage3_body(p0_ref, p1_ref, p2_ref, p3_ref, r0_ref, r1_ref,
                 z0_ref, z1_ref, z2_ref, z3_ref, den_ref,
                 w2a_ref, we_ref, w2b_ref, o_ref):
    w2c = jnp.dot(we_ref[...], w2b_ref[...], preferred_element_type=jnp.float32)
    den = den_ref[...]
    has_msg = den > 0.0
    inv = jnp.where(has_msg, 1.0 / jnp.where(has_msg, den, 1.0), 0.0)
    p = jnp.concatenate(
        [p0_ref[...], p1_ref[...], p2_ref[...], p3_ref[...]], axis=1) * inv
    r = (r0_ref[...] + r1_ref[...]) * inv
    h = jnp.dot(p, w2a_ref[...], preferred_element_type=jnp.float32)
    h = h + jnp.dot(r, w2c, preferred_element_type=jnp.float32)
    z = jnp.concatenate(
        [z0_ref[...], z1_ref[...], z2_ref[...], z3_ref[...]], axis=1)
    o_ref[...] = jnp.where(has_msg, h, z)


def _stage3(ps, rs, zs, denom, W2a, We, W2b):
    zspec = pl.BlockSpec((NROWS, H), lambda i: (i, 0))
    rspec = pl.BlockSpec((NROWS, ED), lambda i: (i, 0))
    return pl.pallas_call(
        _stage3_body,
        grid=(NBLK,),
        in_specs=[
            zspec, zspec, zspec, zspec,
            rspec, rspec,
            zspec, zspec, zspec, zspec,
            pl.BlockSpec((NROWS, 1), lambda i: (i, 0)),
            pl.BlockSpec((D, D), lambda i: (0, 0)),
            pl.BlockSpec((ED, ED), lambda i: (0, 0)),
            pl.BlockSpec((ED, D), lambda i: (0, 0)),
        ],
        out_specs=pl.BlockSpec((NROWS, D), lambda i: (i, 0)),
        out_shape=jax.ShapeDtypeStruct((N, D), jnp.float32),
    )(*ps, *rs, *zs, denom, W2a, We, W2b)


def _f16(v):
    return jnp.full((16,), v, jnp.float32)


def _sc_edge_body(src_hbm, dst_hbm, c_hbm, s1_hbm, s2_hbm,
                  zq0_hbm, zq1_hbm, zq2_hbm, zq3_hbm, ef_hbm,
                  p0_out, p1_out, p2_out, p3_out, r0_out, r1_out, den_out,
                  s1_v, s2_v, src_v, dst_v, ac_v, den_v,
                  g0_v, g1_v, e0_v, e1_v,
                  zrow, vec16, m16,
                  p_sp, r_sp, den_sp, gmax_sp,
                  sem_g0, sem_g1, sem_s0, sem_s1):
    t = lax.axis_index("s")
    cid = lax.axis_index("c")
    base = t * ET

    pltpu.sync_copy(s1_hbm, s1_v)
    pltpu.sync_copy(s2_hbm, s2_v)
    pltpu.sync_copy(src_hbm.at[t], src_v)
    pltpu.sync_copy(dst_hbm.at[t], dst_v)
    pltpu.sync_copy(c_hbm.at[t], ac_v)

    zero16 = _f16(0.0)

    def _zero_g0(r, _):
        for q in range(H // 16):
            g0_v[r, pl.ds(q * 16, 16)] = zero16
        return 0

    def _zero_e0(r, _):
        e0_v[r, pl.ds(0, 16)] = zero16
        return 0

    def _zr(i, _):
        zrow[pl.ds(i * 16, 16)] = zero16
        return 0
    lax.fori_loop(0, 62, _zr, 0)
    zrow[pl.ds(984, 16)] = zero16

    def _zero_p_sp():
        lax.fori_loop(0, C, _zero_g0, 0)
        for k in range(7):
            pltpu.sync_copy(g0_v.at[pl.ds(0, C)],
                            p_sp.at[pl.ds(t * NSL + k * C, C)])
        pltpu.sync_copy(g0_v.at[pl.ds(0, 64)],
                        p_sp.at[pl.ds(t * NSL + 7 * C, 64)])

        @pl.when(t == 15)
        def _():
            pltpu.sync_copy(g0_v.at[pl.ds(0, 16)], p_sp.at[pl.ds(9984, 16)])

    _zero_p_sp()
    lax.fori_loop(0, C, _zero_e0, 0)
    for k in range(7):
        pltpu.sync_copy(e0_v.at[pl.ds(0, C)],
                        r_sp.at[pl.ds(t * NSL + k * C, C)])
    pltpu.sync_copy(e0_v.at[pl.ds(0, 64)],
                    r_sp.at[pl.ds(t * NSL + 7 * C, 64)])

    @pl.when(t == 15)
    def _():
        pltpu.sync_copy(e0_v.at[pl.ds(0, 16)], r_sp.at[pl.ds(9984, 16)])

    @pl.when(t < 10)
    def _():
        pltpu.sync_copy(zrow, den_sp.at[pl.ds(t * 1000, 1000)])

    def _pass_a(j, m):
        for k in range(C // 16):
            sl = pl.ds(k * 16, 16)
            sv = src_v[j, sl]
            dv = dst_v[j, sl]
            g1 = plsc.load_gather(s1_v, [sv])
            g2 = plsc.load_gather(s2_v, [dv])
            a = g1 + ac_v[j, sl] + g2
            e = jnp.where(a > 0.0, a, 0.01 * a)
            ac_v[j, sl] = e
            m = jnp.maximum(m, e)
        return m
    m = lax.fori_loop(0, NCH, _pass_a, _f16(-1e30))

    vec16[pl.ds(0, 16)] = m
    pltpu.sync_copy(vec16, gmax_sp.at[t])
    plsc.subcore_barrier()
    pltpu.sync_copy(gmax_sp, m16)

    def _mred(j, mm):
        return jnp.maximum(mm, m16[j, pl.ds(0, 16)])
    m = lax.fori_loop(0, 16, _mred, _f16(-1e30))
    gv = _f16(jnp.max(m))

    def _pass_b(j, _):
        for k in range(C // 16):
            sl = pl.ds(k * 16, 16)
            ac_v[j, sl] = jnp.exp(ac_v[j, sl] - gv)
        pltpu.sync_copy(ac_v.at[j], den_sp.at[dst_v.at[j]], add=True)
        return 0
    lax.fori_loop(0, NCH, _pass_b, 0)
    plsc.subcore_barrier()

    @pl.when(jnp.logical_and(cid == 0, t < 10))
    def _():
        sl1k = pl.ds(t * 1000, 1000)
        pltpu.sync_copy(den_sp.at[sl1k], den_v.at[sl1k])

    def _bcast_w(j, r):
        jv = jnp.full((16,), j, jnp.int32)
        rv = jnp.full((16,), r, jnp.int32)
        return plsc.load_gather(ac_v, [jv, rv])

    def _scale_rows(j, gref, sref, nq):
        def _r(r4, _):
            for u in range(4):
                r = r4 * 4 + u
                av = _bcast_w(j, r)
                for q in range(nq):
                    sl = pl.ds(q * 16, 16)
                    sref[r, sl] = gref[r, sl] * av
            return 0
        lax.fori_loop(0, C // 4, _r, 0)

    def _heavy(tbl):
        def _gather(j, g, s):
            pltpu.async_copy(tbl.at[src_v.at[j]], g, s)

        def _gwait(g, s):
            pltpu.make_async_copy(tbl.at[src_v.at[0]], g, s).wait()

        def _swait(j, b, s):
            pltpu.make_async_copy(b, p_sp.at[dst_v.at[j]], s).wait()

        def _proc(j, g):
            _scale_rows(j, g, g, H // 16)
            pltpu.sync_copy(g, p_sp.at[dst_v.at[j]], add=True)

        _gather(0, g0_v, sem_g0)

        def _pair(i, _):
            jA = 2 * i
            _gwait(g0_v, sem_g0)
            _gather(jA + 1, g1_v, sem_g1)
            _proc(jA, g0_v)

            _gwait(g1_v, sem_g1)
            _gather(jA + 2, g0_v, sem_g0)
            _proc(jA + 1, g1_v)
            return 0
        lax.fori_loop(0, NCH // 2, _pair, 0)

        jT = NCH - 1
        _gwait(g0_v, sem_g0)
        _proc(jT, g0_v)

    def _r_phase(r_lo, r_cnt):
        def _eload(j, e, s):
            pltpu.async_copy(ef_hbm.at[pl.ds(base + j * C, C)], e, s)

        def _ewait(e, s):
            pltpu.make_async_copy(ef_hbm.at[pl.ds(base, C)], e, s).wait()

        def _rswait(j, e, s):
            pltpu.make_async_copy(e, r_sp.at[dst_v.at[j]], s).wait()

        def _rproc(j, e):
            _scale_rows(j, e, e, 1)
            pltpu.sync_copy(e, r_sp.at[dst_v.at[j]], add=True)

        _eload(r_lo, e0_v, sem_g0)

        def _rpair(i, _):
            jA = r_lo + 2 * i
            _ewait(e0_v, sem_g0)
            _eload(jA + 1, e1_v, sem_g1)
            _rproc(jA, e0_v)

            _ewait(e1_v, sem_g1)

            @pl.when(jA + 2 < r_lo + r_cnt)
            def _():
                _eload(jA + 2, e0_v, sem_g0)
            _rproc(jA + 1, e1_v)
            return 0
        lax.fori_loop(0, r_cnt // 2, _rpair, 0)

        if r_cnt % 2 == 1:
            jT = r_lo + r_cnt - 1
            _ewait(e0_v, sem_g0)
            _rproc(jT, e0_v)

    def _write_p(pout):
        rows = pl.ds(t * NSL, NSL)
        pltpu.sync_copy(p_sp.at[rows], pout.at[rows])

        @pl.when(t == 15)
        def _():
            tail = pl.ds(9984, 16)
            pltpu.sync_copy(p_sp.at[tail], pout.at[tail])

    def _write_r(rout):
        rows_ = pl.ds(t * NSL, NSL)
        pltpu.sync_copy(r_sp.at[rows_], rout.at[rows_])

        @pl.when(t == 15)
        def _():
            tail_ = pl.ds(9984, 16)
            pltpu.sync_copy(r_sp.at[tail_], rout.at[tail_])

    @pl.when(cid == 0)
    def _():
        _heavy(zq0_hbm)
        _r_phase(0, 62)

    @pl.when(cid == 1)
    def _():
        _heavy(zq2_hbm)
        _r_phase(62, NCH - 62)

    plsc.subcore_barrier()

    @pl.when(cid == 0)
    def _():
        _write_p(p0_out)
        _write_r(r0_out)

    @pl.when(jnp.logical_and(cid == 0, t < 10))
    def _():
        sl1k = pl.ds(t * 1000, 1000)
        pltpu.sync_copy(den_v.at[sl1k], den_out.at[sl1k])

    @pl.when(cid == 1)
    def _():
        _write_p(p2_out)
        _write_r(r1_out)

    _zero_p_sp()
    plsc.subcore_barrier()

    @pl.when(cid == 0)
    def _():
        _heavy(zq1_hbm)

    @pl.when(cid == 1)
    def _():
        _heavy(zq3_hbm)

    plsc.subcore_barrier()

    @pl.when(cid == 0)
    def _():
        _write_p(p1_out)

    @pl.when(cid == 1)
    def _():
        _write_p(p3_out)


@functools.partial(
    pl.kernel,
    mesh=plsc.VectorSubcoreMesh(core_axis_name="c", subcore_axis_name="s"),
    compiler_params=pltpu.CompilerParams(
        needs_layout_passes=False, use_tc_tiling_on_sc=False),
    out_type=[
        jax.ShapeDtypeStruct((N, H), jnp.float32),
        jax.ShapeDtypeStruct((N, H), jnp.float32),
        jax.ShapeDtypeStruct((N, H), jnp.float32),
        jax.ShapeDtypeStruct((N, H), jnp.float32),
        jax.ShapeDtypeStruct((N, ED), jnp.float32),
        jax.ShapeDtypeStruct((N, ED), jnp.float32),
        jax.ShapeDtypeStruct((N,), jnp.float32),
    ],
    scratch_types=[
        pltpu.VMEM((N,), jnp.float32),
        pltpu.VMEM((N,), jnp.float32),
        pltpu.VMEM((NCH, C), jnp.int32),
        pltpu.VMEM((NCH, C), jnp.int32),
        pltpu.VMEM((NCH, C), jnp.float32),
        pltpu.VMEM((N,), jnp.float32),
        pltpu.VMEM((C, H), jnp.float32),
        pltpu.VMEM((C, H), jnp.float32),
        pltpu.VMEM((C, ED), jnp.float32),
        pltpu.VMEM((C, ED), jnp.float32),
        pltpu.VMEM((1000,), jnp.float32),
        pltpu.VMEM((16,), jnp.float32),
        pltpu.VMEM((16, 16), jnp.float32),
        pltpu.VMEM_SHARED((N, H), jnp.float32),
        pltpu.VMEM_SHARED((N, ED), jnp.float32),
        pltpu.VMEM_SHARED((N,), jnp.float32),
        pltpu.VMEM_SHARED((16, 16), jnp.float32),
        pltpu.SemaphoreType.DMA,
        pltpu.SemaphoreType.DMA,
        pltpu.SemaphoreType.DMA,
        pltpu.SemaphoreType.DMA,
    ],
)
def _sc_edge(src, dst, c, s1, s2, zq0, zq1, zq2, zq3, ef,
             p0, p1, p2, p3, r0, r1, den, *scratch):
    _sc_edge_body(src, dst, c, s1, s2, zq0, zq1, zq2, zq3, ef,
                  p0, p1, p2, p3, r0, r1, den, *scratch)


def kernel(node_features, edges_features, edge_index, Wn, We, Wa, W2):
    src = edge_index[0]
    dst = edge_index[1]

    wa_pair = jnp.zeros((D, 128), jnp.float32)
    wa_pair = wa_pair.at[:, 0].set(Wa[:D, 0]).at[:, 1].set(Wa[D + ED:, 0])
    zq0, zq1, zq2, zq3, s_pair = _stage1(node_features, Wn, wa_pair)
    s1 = s_pair[:, 0]
    s2 = s_pair[:, 1]

    ef3 = edges_features.reshape(E // 128, 128, ED)
    c = _stage_c(ef3, We, Wa[D:D + ED]).reshape(E)

    src3 = src.reshape(16, NCH, C)
    dst3 = dst.reshape(16, NCH, C)
    c3 = c.reshape(16, NCH, C)
    p0, p1, p2, p3, r0, r1, denom = _sc_edge(src3, dst3, c3, s1, s2,
                                             zq0, zq1, zq2, zq3,
                                             edges_features)

    return _stage3((p0, p1, p2, p3), (r0, r1), (zq0, zq1, zq2, zq3),
                   denom[:, None], W2[:D], We, W2[D:])


# speedup vs baseline: 8.3524x; 1.0083x over previous
import functools
import jax
import jax.numpy as jnp
from jax import lax
from jax.experimental import pallas as pl
from jax.experimental.pallas import tpu as pltpu
from jax.experimental.pallas import tpu_sc as plsc

N = 10000
E = 160000
D = 256
ED = 16

NBLK = 10
NROWS = N // NBLK
EROWS = 128
EBLK = (E // 128 + EROWS - 1) // EROWS

ET = E // 16
C = 80
NCH = ET // C
NSL = 624
VSTEPS = ET // 16
H = 64


def _stage1_body(x_ref, wn_ref, wa_ref, z0_ref, z1_ref, z2_ref, z3_ref, s_ref):
    z = jnp.dot(x_ref[...], wn_ref[...], preferred_element_type=jnp.float32)
    z0_ref[...] = z[:, 0:64]
    z1_ref[...] = z[:, 64:128]
    z2_ref[...] = z[:, 128:192]
    z3_ref[...] = z[:, 192:256]
    s_ref[...] = jnp.dot(z, wa_ref[...], preferred_element_type=jnp.float32)


def _stage1(x, Wn, wa_pair):
    zspec = pl.BlockSpec((NROWS, H), lambda i: (i, 0))
    zshape = jax.ShapeDtypeStruct((N, H), jnp.float32)
    return pl.pallas_call(
        _stage1_body,
        grid=(NBLK,),
        in_specs=[
            pl.BlockSpec((NROWS, D), lambda i: (i, 0)),
            pl.BlockSpec((D, D), lambda i: (0, 0)),
            pl.BlockSpec((D, 128), lambda i: (0, 0)),
        ],
        out_specs=[zspec, zspec, zspec, zspec,
                   pl.BlockSpec((NROWS, 128), lambda i: (i, 0))],
        out_shape=[zshape, zshape, zshape, zshape,
                   jax.ShapeDtypeStruct((N, 128), jnp.float32)],
    )(x, Wn, wa_pair)


def _stage_c_body(ef_ref, we_ref, wam_ref, c_ref):
    w_ec = jnp.dot(we_ref[...], wam_ref[...], preferred_element_type=jnp.float32)
    ef = ef_ref[...]
    c_ref[...] = jnp.sum(ef * w_ec[:, 0][None, None, :], axis=-1)


def _stage_c(ef3, We, wa_mid):
    return pl.pallas_call(
        _stage_c_body,
        grid=(EBLK,),
        in_specs=[
            pl.BlockSpec((EROWS, 128, ED), lambda i: (i, 0, 0)),
            pl.BlockSpec((ED, ED), lambda i: (0, 0)),
            pl.BlockSpec((ED, 1), lambda i: (0, 0)),
        ],
        out_specs=pl.BlockSpec((EROWS, 128), lambda i: (i, 0)),
        out_shape=jax.ShapeDtypeStruct((E // 128, 128), jnp.float32),
    )(ef3, We, wa_mid)


def _stage3_body(p0_ref, p1_ref, p2_ref, p3_ref, r0_ref, r1_ref,
                 z0_ref, z1_ref, z2_ref, z3_ref, den_ref,
                 w2a_ref, we_ref, w2b_ref, o_ref):
    w2c = jnp.dot(we_ref[...], w2b_ref[...], preferred_element_type=jnp.float32)
    den = den_ref[...]
    has_msg = den > 0.0
    inv = jnp.where(has_msg, 1.0 / jnp.where(has_msg, den, 1.0), 0.0)
    p = jnp.concatenate(
        [p0_ref[...], p1_ref[...], p2_ref[...], p3_ref[...]], axis=1) * inv
    r = (r0_ref[...] + r1_ref[...]) * inv
    h = jnp.dot(p, w2a_ref[...], preferred_element_type=jnp.float32)
    h = h + jnp.dot(r, w2c, preferred_element_type=jnp.float32)
    z = jnp.concatenate(
        [z0_ref[...], z1_ref[...], z2_ref[...], z3_ref[...]], axis=1)
    o_ref[...] = jnp.where(has_msg, h, z)


def _stage3(ps, rs, zs, denom, W2a, We, W2b):
    zspec = pl.BlockSpec((NROWS, H), lambda i: (i, 0))
    rspec = pl.BlockSpec((NROWS, ED), lambda i: (i, 0))
    return pl.pallas_call(
        _stage3_body,
        grid=(NBLK,),
        in_specs=[
            zspec, zspec, zspec, zspec,
            rspec, rspec,
            zspec, zspec, zspec, zspec,
            pl.BlockSpec((NROWS, 1), lambda i: (i, 0)),
            pl.BlockSpec((D, D), lambda i: (0, 0)),
            pl.BlockSpec((ED, ED), lambda i: (0, 0)),
            pl.BlockSpec((ED, D), lambda i: (0, 0)),
        ],
        out_specs=pl.BlockSpec((NROWS, D), lambda i: (i, 0)),
        out_shape=jax.ShapeDtypeStruct((N, D), jnp.float32),
    )(*ps, *rs, *zs, denom, W2a, We, W2b)


def _f16(v):
    return jnp.full((16,), v, jnp.float32)


def _sc_edge_body(src_hbm, dst_hbm, c_hbm, s1_hbm, s2_hbm,
                  zq0_hbm, zq1_hbm, zq2_hbm, zq3_hbm, ef_hbm,
                  p0_out, p1_out, p2_out, p3_out, r0_out, r1_out, den_out,
                  s1_v, s2_v, src_v, dst_v, ac_v, den_v,
                  g0_v, g1_v, g2_v, e0_v, e1_v,
                  zrow, vec16, m16,
                  p_sp, r_sp, den_sp, gmax_sp,
                  sem_g0, sem_g1, sem_g2, sem_s0, sem_s1, sem_s2):
    t = lax.axis_index("s")
    cid = lax.axis_index("c")
    base = t * ET

    pltpu.sync_copy(s1_hbm, s1_v)
    pltpu.sync_copy(s2_hbm, s2_v)
    pltpu.sync_copy(src_hbm.at[t], src_v)
    pltpu.sync_copy(dst_hbm.at[t], dst_v)
    pltpu.sync_copy(c_hbm.at[t], ac_v)

    zero16 = _f16(0.0)

    def _zero_g0(r, _):
        for q in range(H // 16):
            g0_v[r, pl.ds(q * 16, 16)] = zero16
        return 0

    def _zero_e0(r, _):
        e0_v[r, pl.ds(0, 16)] = zero16
        return 0

    def _zr(i, _):
        zrow[pl.ds(i * 16, 16)] = zero16
        return 0
    lax.fori_loop(0, 62, _zr, 0)
    zrow[pl.ds(984, 16)] = zero16

    def _zero_p_sp():
        lax.fori_loop(0, C, _zero_g0, 0)
        for k in range(7):
            pltpu.sync_copy(g0_v.at[pl.ds(0, C)],
                            p_sp.at[pl.ds(t * NSL + k * C, C)])
        pltpu.sync_copy(g0_v.at[pl.ds(0, 64)],
                        p_sp.at[pl.ds(t * NSL + 7 * C, 64)])

        @pl.when(t == 15)
        def _():
            pltpu.sync_copy(g0_v.at[pl.ds(0, 16)], p_sp.at[pl.ds(9984, 16)])

    _zero_p_sp()
    lax.fori_loop(0, C, _zero_e0, 0)
    for k in range(7):
        pltpu.sync_copy(e0_v.at[pl.ds(0, C)],
                        r_sp.at[pl.ds(t * NSL + k * C, C)])
    pltpu.sync_copy(e0_v.at[pl.ds(0, 64)],
                    r_sp.at[pl.ds(t * NSL + 7 * C, 64)])

    @pl.when(t == 15)
    def _():
        pltpu.sync_copy(e0_v.at[pl.ds(0, 16)], r_sp.at[pl.ds(9984, 16)])

    @pl.when(t < 10)
    def _():
        pltpu.sync_copy(zrow, den_sp.at[pl.ds(t * 1000, 1000)])

    def _pass_a(j, m):
        for k in range(C // 16):
            sl = pl.ds(k * 16, 16)
            sv = src_v[j, sl]
            dv = dst_v[j, sl]
            g1 = plsc.load_gather(s1_v, [sv])
            g2 = plsc.load_gather(s2_v, [dv])
            a = g1 + ac_v[j, sl] + g2
            e = jnp.where(a > 0.0, a, 0.01 * a)
            ac_v[j, sl] = e
            m = jnp.maximum(m, e)
        return m
    m = lax.fori_loop(0, NCH, _pass_a, _f16(-1e30))

    vec16[pl.ds(0, 16)] = m
    pltpu.sync_copy(vec16, gmax_sp.at[pl.ds(t * 16, 16)])
    plsc.subcore_barrier()
    pltpu.sync_copy(gmax_sp, m16)

    def _mred(j, mm):
        return jnp.maximum(mm, m16[pl.ds(j * 16, 16)])
    m = lax.fori_loop(0, 16, _mred, _f16(-1e30))
    gv = _f16(jnp.max(m))

    def _pass_b(j, _):
        for k in range(C // 16):
            sl = pl.ds(k * 16, 16)
            ac_v[j, sl] = jnp.exp(ac_v[j, sl] - gv)
        pltpu.sync_copy(ac_v.at[j], den_sp.at[dst_v.at[j]], add=True)
        return 0
    lax.fori_loop(0, NCH, _pass_b, 0)
    plsc.subcore_barrier()

    @pl.when(jnp.logical_and(cid == 0, t < 10))
    def _():
        sl1k = pl.ds(t * 1000, 1000)
        pltpu.sync_copy(den_sp.at[sl1k], den_v.at[sl1k])

    def _bcast_w(j, r):
        jv = jnp.full((16,), j, jnp.int32)
        rv = jnp.full((16,), r, jnp.int32)
        return plsc.load_gather(ac_v, [jv, rv])

    def _scale_rows(j, gref, sref, nq):
        def _r(r4, _):
            for u in range(4):
                r = r4 * 4 + u
                av = _bcast_w(j, r)
                for q in range(nq):
                    sl = pl.ds(q * 16, 16)
                    sref[r, sl] = gref[r, sl] * av
            return 0
        lax.fori_loop(0, C // 4, _r, 0)

    def _heavy(tbl):
        bufs = (g0_v, g1_v, g2_v)
        gsems = (sem_g0, sem_g1, sem_g2)
        ssems = (sem_s0, sem_s1, sem_s2)

        def _gather(j, g, s):
            pltpu.async_copy(tbl.at[src_v.at[j]], g, s)

        def _gwait(g, s):
            pltpu.make_async_copy(tbl.at[src_v.at[0]], g, s).wait()

        def _swait(j, b, s):
            pltpu.make_async_copy(b, p_sp.at[dst_v.at[j]], s).wait()

        def _step(j, k, first, prefetch):
            g, gs, ss = bufs[k], gsems[k], ssems[k]
            kp = (k + 2) % 3
            _gwait(g, gs)
            if prefetch:
                if first:
                    _gather(j + 2, bufs[kp], gsems[kp])
                else:
                    _swait(j - 1, bufs[kp], ssems[kp])
                    _gather(j + 2, bufs[kp], gsems[kp])
            _scale_rows(j, g, g, H // 16)
            pltpu.async_copy(g, p_sp.at[dst_v.at[j]], ss, add=True)

        _gather(0, g0_v, sem_g0)
        _gather(1, g1_v, sem_g1)
        _step(0, 0, True, True)

        def _triple(i, _):
            j0 = 3 * i
            _step(j0 + 1, 1, False, True)
            _step(j0 + 2, 2, False, True)
            _step(j0 + 3, 0, False, True)
            return 0
        lax.fori_loop(0, (NCH - 5) // 3, _triple, 0)
        _step(NCH - 4, 1, False, True)
        _step(NCH - 3, 2, False, True)
        _step(NCH - 2, 0, False, False)
        _step(NCH - 1, 1, False, False)
        for dj in (NCH - 3, NCH - 2, NCH - 1):
            _swait(dj, bufs[dj % 3], ssems[dj % 3])

    def _r_phase(r_lo, r_cnt):
        def _eload(j, e, s):
            pltpu.async_copy(ef_hbm.at[pl.ds(base + j * C, C)], e, s)

        def _ewait(e, s):
            pltpu.make_async_copy(ef_hbm.at[pl.ds(base, C)], e, s).wait()

        def _rswait(j, e, s):
            pltpu.make_async_copy(e, r_sp.at[dst_v.at[j]], s).wait()

        def _rproc(j, e):
            _scale_rows(j, e, e, 1)
            pltpu.sync_copy(e, r_sp.at[dst_v.at[j]], add=True)

        _eload(r_lo, e0_v, sem_g0)

        def _rpair(i, _):
            jA = r_lo + 2 * i
            _ewait(e0_v, sem_g0)
            _eload(jA + 1, e1_v, sem_g1)
            _rproc(jA, e0_v)

            _ewait(e1_v, sem_g1)

            @pl.when(jA + 2 < r_lo + r_cnt)
            def _():
                _eload(jA + 2, e0_v, sem_g0)
            _rproc(jA + 1, e1_v)
            return 0
        lax.fori_loop(0, r_cnt // 2, _rpair, 0)

        if r_cnt % 2 == 1:
            jT = r_lo + r_cnt - 1
            _ewait(e0_v, sem_g0)
            _rproc(jT, e0_v)

    def _write_p(pout):
        rows = pl.ds(t * NSL, NSL)
        pltpu.sync_copy(p_sp.at[rows], pout.at[rows])

        @pl.when(t == 15)
        def _():
            tail = pl.ds(9984, 16)
            pltpu.sync_copy(p_sp.at[tail], pout.at[tail])

    def _write_r(rout):
        rows_ = pl.ds(t * NSL, NSL)
        pltpu.sync_copy(r_sp.at[rows_], rout.at[rows_])

        @pl.when(t == 15)
        def _():
            tail_ = pl.ds(9984, 16)
            pltpu.sync_copy(r_sp.at[tail_], rout.at[tail_])

    @pl.when(cid == 0)
    def _():
        _heavy(zq0_hbm)
        _r_phase(0, 62)

    @pl.when(cid == 1)
    def _():
        _heavy(zq2_hbm)
        _r_phase(62, NCH - 62)

    plsc.subcore_barrier()

    @pl.when(cid == 0)
    def _():
        _write_p(p0_out)
        _write_r(r0_out)

    @pl.when(jnp.logical_and(cid == 0, t < 10))
    def _():
        sl1k = pl.ds(t * 1000, 1000)
        pltpu.sync_copy(den_v.at[sl1k], den_out.at[sl1k])

    @pl.when(cid == 1)
    def _():
        _write_p(p2_out)
        _write_r(r1_out)

    _zero_p_sp()
    plsc.subcore_barrier()

    @pl.when(cid == 0)
    def _():
        _heavy(zq1_hbm)

    @pl.when(cid == 1)
    def _():
        _heavy(zq3_hbm)

    plsc.subcore_barrier()

    @pl.when(cid == 0)
    def _():
        _write_p(p1_out)

    @pl.when(cid == 1)
    def _():
        _write_p(p3_out)


@functools.partial(
    pl.kernel,
    mesh=plsc.VectorSubcoreMesh(core_axis_name="c", subcore_axis_name="s"),
    compiler_params=pltpu.CompilerParams(
        needs_layout_passes=False, use_tc_tiling_on_sc=False),
    out_type=[
        jax.ShapeDtypeStruct((N, H), jnp.float32),
        jax.ShapeDtypeStruct((N, H), jnp.float32),
        jax.ShapeDtypeStruct((N, H), jnp.float32),
        jax.ShapeDtypeStruct((N, H), jnp.float32),
        jax.ShapeDtypeStruct((N, ED), jnp.float32),
        jax.ShapeDtypeStruct((N, ED), jnp.float32),
        jax.ShapeDtypeStruct((N,), jnp.float32),
    ],
    scratch_types=[
        pltpu.VMEM((N,), jnp.float32),
        pltpu.VMEM((N,), jnp.float32),
        pltpu.VMEM((NCH, C), jnp.int32),
        pltpu.VMEM((NCH, C), jnp.int32),
        pltpu.VMEM((NCH, C), jnp.float32),
        pltpu.VMEM((N,), jnp.float32),
        pltpu.VMEM((C, H), jnp.float32),
        pltpu.VMEM((C, H), jnp.float32),
        pltpu.VMEM((C, H), jnp.float32),
        pltpu.VMEM((C, ED), jnp.float32),
        pltpu.VMEM((C, ED), jnp.float32),
        pltpu.VMEM((1000,), jnp.float32),
        pltpu.VMEM((16,), jnp.float32),
        pltpu.VMEM((256,), jnp.float32),
        pltpu.VMEM_SHARED((N, H), jnp.float32),
        pltpu.VMEM_SHARED((N, ED), jnp.float32),
        pltpu.VMEM_SHARED((N,), jnp.float32),
        pltpu.VMEM_SHARED((256,), jnp.float32),
        pltpu.SemaphoreType.DMA,
        pltpu.SemaphoreType.DMA,
        pltpu.SemaphoreType.DMA,
        pltpu.SemaphoreType.DMA,
        pltpu.SemaphoreType.DMA,
        pltpu.SemaphoreType.DMA,
    ],
)
def _sc_edge(src, dst, c, s1, s2, zq0, zq1, zq2, zq3, ef,
             p0, p1, p2, p3, r0, r1, den, *scratch):
    _sc_edge_body(src, dst, c, s1, s2, zq0, zq1, zq2, zq3, ef,
                  p0, p1, p2, p3, r0, r1, den, *scratch)


def kernel(node_features, edges_features, edge_index, Wn, We, Wa, W2):
    src = edge_index[0]
    dst = edge_index[1]

    wa_pair = jnp.zeros((D, 128), jnp.float32)
    wa_pair = wa_pair.at[:, 0].set(Wa[:D, 0]).at[:, 1].set(Wa[D + ED:, 0])
    zq0, zq1, zq2, zq3, s_pair = _stage1(node_features, Wn, wa_pair)
    s1 = s_pair[:, 0]
    s2 = s_pair[:, 1]

    ef3 = edges_features.reshape(E // 128, 128, ED)
    c = _stage_c(ef3, We, Wa[D:D + ED]).reshape(E)

    src3 = src.reshape(16, NCH, C)
    dst3 = dst.reshape(16, NCH, C)
    c3 = c.reshape(16, NCH, C)
    p0, p1, p2, p3, r0, r1, denom = _sc_edge(src3, dst3, c3, s1, s2,
                                             zq0, zq1, zq2, zq3,
                                             edges_features)

    return _stage3((p0, p1, p2, p3), (r0, r1), (zq0, zq1, zq2, zq3),
                   denom[:, None], W2[:D], We, W2[D:])


# speedup vs baseline: 8.4984x; 1.0175x over previous
import functools
import jax
import jax.numpy as jnp
from jax import lax
from jax.experimental import pallas as pl
from jax.experimental.pallas import tpu as pltpu
from jax.experimental.pallas import tpu_sc as plsc

N = 10000
E = 160000
D = 256
ED = 16

NBLK = 10
NROWS = N // NBLK
EROWS = 128
EBLK = (E // 128 + EROWS - 1) // EROWS

ET = E // 16
C = 80
NCH = ET // C
NSL = 624
VSTEPS = ET // 16
H = 64


def _stage1_body(x_ref, wn_ref, wa_ref, z0_ref, z1_ref, z2_ref, z3_ref, s_ref):
    z = jnp.dot(x_ref[...], wn_ref[...], preferred_element_type=jnp.float32)
    z0_ref[...] = z[:, 0:64]
    z1_ref[...] = z[:, 64:128]
    z2_ref[...] = z[:, 128:192]
    z3_ref[...] = z[:, 192:256]
    s_ref[...] = jnp.dot(z, wa_ref[...], preferred_element_type=jnp.float32)


def _stage1(x, Wn, wa_pair):
    zspec = pl.BlockSpec((NROWS, H), lambda i: (i, 0))
    zshape = jax.ShapeDtypeStruct((N, H), jnp.float32)
    return pl.pallas_call(
        _stage1_body,
        grid=(NBLK,),
        in_specs=[
            pl.BlockSpec((NROWS, D), lambda i: (i, 0)),
            pl.BlockSpec((D, D), lambda i: (0, 0)),
            pl.BlockSpec((D, 128), lambda i: (0, 0)),
        ],
        out_specs=[zspec, zspec, zspec, zspec,
                   pl.BlockSpec((NROWS, 128), lambda i: (i, 0))],
        out_shape=[zshape, zshape, zshape, zshape,
                   jax.ShapeDtypeStruct((N, 128), jnp.float32)],
    )(x, Wn, wa_pair)


def _stage_c_body(ef_ref, we_ref, wam_ref, c_ref):
    w_ec = jnp.dot(we_ref[...], wam_ref[...], preferred_element_type=jnp.float32)
    ef = ef_ref[...]
    c_ref[...] = jnp.sum(ef * w_ec[:, 0][None, None, :], axis=-1)


CROWS = 200


def _stage_c(ef3, We, wa_mid):
    return pl.pallas_call(
        _stage_c_body,
        grid=(EBLK,),
        in_specs=[
            pl.BlockSpec((CROWS, C, ED), lambda i: (i, 0, 0)),
            pl.BlockSpec((ED, ED), lambda i: (0, 0)),
            pl.BlockSpec((ED, 1), lambda i: (0, 0)),
        ],
        out_specs=pl.BlockSpec((CROWS, C), lambda i: (i, 0)),
        out_shape=jax.ShapeDtypeStruct((E // C, C), jnp.float32),
    )(ef3, We, wa_mid)


def _stage3_body(p0_ref, p1_ref, p2_ref, p3_ref, r0_ref, r1_ref,
                 z0_ref, z1_ref, z2_ref, z3_ref, den_ref,
                 w2a_ref, we_ref, w2b_ref, o_ref):
    w2c = jnp.dot(we_ref[...], w2b_ref[...], preferred_element_type=jnp.float32)
    den = den_ref[...]
    has_msg = den > 0.0
    inv = jnp.where(has_msg, 1.0 / jnp.where(has_msg, den, 1.0), 0.0)
    p = jnp.concatenate(
        [p0_ref[...], p1_ref[...], p2_ref[...], p3_ref[...]], axis=1) * inv
    r = (r0_ref[...] + r1_ref[...]) * inv
    h = jnp.dot(p, w2a_ref[...], preferred_element_type=jnp.float32)
    h = h + jnp.dot(r, w2c, preferred_element_type=jnp.float32)
    z = jnp.concatenate(
        [z0_ref[...], z1_ref[...], z2_ref[...], z3_ref[...]], axis=1)
    o_ref[...] = jnp.where(has_msg, h, z)


def _stage3(ps, rs, zs, denom, W2a, We, W2b):
    zspec = pl.BlockSpec((NROWS, H), lambda i: (i, 0))
    rspec = pl.BlockSpec((NROWS, ED), lambda i: (i, 0))
    return pl.pallas_call(
        _stage3_body,
        grid=(NBLK,),
        in_specs=[
            zspec, zspec, zspec, zspec,
            rspec, rspec,
            zspec, zspec, zspec, zspec,
            pl.BlockSpec((NROWS, 1), lambda i: (i, 0)),
            pl.BlockSpec((D, D), lambda i: (0, 0)),
            pl.BlockSpec((ED, ED), lambda i: (0, 0)),
            pl.BlockSpec((ED, D), lambda i: (0, 0)),
        ],
        out_specs=pl.BlockSpec((NROWS, D), lambda i: (i, 0)),
        out_shape=jax.ShapeDtypeStruct((N, D), jnp.float32),
    )(*ps, *rs, *zs, denom, W2a, We, W2b)


def _f16(v):
    return jnp.full((16,), v, jnp.float32)


def _sc_edge_body(ei_hbm, c_hbm, s1_hbm, s2_hbm,
                  zq0_hbm, zq1_hbm, zq2_hbm, zq3_hbm, ef_hbm,
                  p0_out, p1_out, p2_out, p3_out, r0_out, r1_out, den_out,
                  s1_v, s2_v, src_v, dst_v, ac_v, den_v,
                  g0_v, g1_v, g2_v, e0_v, e1_v,
                  zrow, vec16, m16,
                  p_sp, r_sp, den_sp, gmax_sp,
                  sem_g0, sem_g1, sem_g2, sem_s0, sem_s1, sem_s2):
    t = lax.axis_index("s")
    cid = lax.axis_index("c")
    base = t * ET

    pltpu.sync_copy(s1_hbm, s1_v)
    pltpu.sync_copy(s2_hbm, s2_v)
    pltpu.sync_copy(ei_hbm.at[0, t], src_v)
    pltpu.sync_copy(ei_hbm.at[1, t], dst_v)
    pltpu.sync_copy(c_hbm.at[t], ac_v)

    zero16 = _f16(0.0)

    def _zero_g0(r, _):
        for q in range(H // 16):
            g0_v[r, pl.ds(q * 16, 16)] = zero16
        return 0

    def _zero_e0(r, _):
        e0_v[r, pl.ds(0, 16)] = zero16
        return 0

    def _zr(i, _):
        zrow[pl.ds(i * 16, 16)] = zero16
        return 0
    lax.fori_loop(0, 62, _zr, 0)
    zrow[pl.ds(984, 16)] = zero16

    def _zero_p_sp():
        lax.fori_loop(0, C, _zero_g0, 0)
        for k in range(7):
            pltpu.sync_copy(g0_v.at[pl.ds(0, C)],
                            p_sp.at[pl.ds(t * NSL + k * C, C)])
        pltpu.sync_copy(g0_v.at[pl.ds(0, 64)],
                        p_sp.at[pl.ds(t * NSL + 7 * C, 64)])

        @pl.when(t == 15)
        def _():
            pltpu.sync_copy(g0_v.at[pl.ds(0, 16)], p_sp.at[pl.ds(9984, 16)])

    _zero_p_sp()
    lax.fori_loop(0, C, _zero_e0, 0)
    for k in range(7):
        pltpu.sync_copy(e0_v.at[pl.ds(0, C)],
                        r_sp.at[pl.ds(t * NSL + k * C, C)])
    pltpu.sync_copy(e0_v.at[pl.ds(0, 64)],
                    r_sp.at[pl.ds(t * NSL + 7 * C, 64)])

    @pl.when(t == 15)
    def _():
        pltpu.sync_copy(e0_v.at[pl.ds(0, 16)], r_sp.at[pl.ds(9984, 16)])

    @pl.when(t < 10)
    def _():
        pltpu.sync_copy(zrow, den_sp.at[pl.ds(t * 1000, 1000)])

    def _pass_a(j, m):
        for k in range(C // 16):
            sl = pl.ds(k * 16, 16)
            sv = src_v[j, sl]
            dv = dst_v[j, sl]
            g1 = plsc.load_gather(s1_v, [sv])
            g2 = plsc.load_gather(s2_v, [dv])
            a = g1 + ac_v[j, sl] + g2
            e = jnp.where(a > 0.0, a, 0.01 * a)
            ac_v[j, sl] = e
            m = jnp.maximum(m, e)
        return m
    m = lax.fori_loop(0, NCH, _pass_a, _f16(-1e30))

    vec16[pl.ds(0, 16)] = m
    pltpu.sync_copy(vec16, gmax_sp.at[pl.ds(t * 16, 16)])
    plsc.subcore_barrier()
    pltpu.sync_copy(gmax_sp, m16)

    def _mred(j, mm):
        return jnp.maximum(mm, m16[pl.ds(j * 16, 16)])
    m = lax.fori_loop(0, 16, _mred, _f16(-1e30))
    gv = _f16(jnp.max(m))

    def _pass_b(j, _):
        for k in range(C // 16):
            sl = pl.ds(k * 16, 16)
            ac_v[j, sl] = jnp.exp(ac_v[j, sl] - gv)
        pltpu.async_copy(ac_v.at[j], den_sp.at[dst_v.at[j]], sem_s0,
                         add=True)

        @pl.when(j >= 16)
        def _():
            pltpu.make_async_copy(ac_v.at[j], den_sp.at[dst_v.at[j]],
                                  sem_s0).wait()
        return 0
    lax.fori_loop(0, NCH, _pass_b, 0)

    def _pass_b_drain(i, _):
        pltpu.make_async_copy(ac_v.at[i], den_sp.at[dst_v.at[i]],
                              sem_s0).wait()
        return 0
    lax.fori_loop(0, 16, _pass_b_drain, 0)
    plsc.subcore_barrier()

    @pl.when(jnp.logical_and(cid == 0, t < 10))
    def _():
        sl1k = pl.ds(t * 1000, 1000)
        pltpu.sync_copy(den_sp.at[sl1k], den_v.at[sl1k])

    def _bcast_w(j, r):
        jv = jnp.full((16,), j, jnp.int32)
        rv = jnp.full((16,), r, jnp.int32)
        return plsc.load_gather(ac_v, [jv, rv])

    def _scale_rows(j, gref, sref, nq):
        def _r(r4, _):
            for u in range(4):
                r = r4 * 4 + u
                av = _bcast_w(j, r)
                for q in range(nq):
                    sl = pl.ds(q * 16, 16)
                    sref[r, sl] = gref[r, sl] * av
            return 0
        lax.fori_loop(0, C // 4, _r, 0)

    def _heavy(tbl):
        bufs = (g0_v, g1_v, g2_v)
        gsems = (sem_g0, sem_g1, sem_g2)
        ssems = (sem_s0, sem_s1, sem_s2)

        def _gather(j, g, s):
            pltpu.async_copy(tbl.at[src_v.at[j]], g, s)

        def _gwait(g, s):
            pltpu.make_async_copy(tbl.at[src_v.at[0]], g, s).wait()

        def _swait(j, b, s):
            pltpu.make_async_copy(b, p_sp.at[dst_v.at[j]], s).wait()

        def _step(j, k, first, prefetch):
            g, gs, ss = bufs[k], gsems[k], ssems[k]
            kp = (k + 2) % 3
            _gwait(g, gs)
            if prefetch:
                if first:
                    _gather(j + 2, bufs[kp], gsems[kp])
                else:
                    _swait(j - 1, bufs[kp], ssems[kp])
                    _gather(j + 2, bufs[kp], gsems[kp])
            _scale_rows(j, g, g, H // 16)
            pltpu.async_copy(g, p_sp.at[dst_v.at[j]], ss, add=True)

        _gather(0, g0_v, sem_g0)
        _gather(1, g1_v, sem_g1)
        _step(0, 0, True, True)

        def _triple(i, _):
            j0 = 3 * i
            _step(j0 + 1, 1, False, True)
            _step(j0 + 2, 2, False, True)
            _step(j0 + 3, 0, False, True)
            return 0
        lax.fori_loop(0, (NCH - 5) // 3, _triple, 0)
        _step(NCH - 4, 1, False, True)
        _step(NCH - 3, 2, False, True)
        _step(NCH - 2, 0, False, False)
        _step(NCH - 1, 1, False, False)
        for dj in (NCH - 3, NCH - 2, NCH - 1):
            _swait(dj, bufs[dj % 3], ssems[dj % 3])

    def _r_phase(r_lo, r_cnt):
        def _eload(j, e, s):
            pltpu.async_copy(ef_hbm.at[pl.ds(base + j * C, C)], e, s)

        def _ewait(e, s):
            pltpu.make_async_copy(ef_hbm.at[pl.ds(base, C)], e, s).wait()

        def _rswait(j, e, s):
            pltpu.make_async_copy(e, r_sp.at[dst_v.at[j]], s).wait()

        def _rproc(j, e):
            _scale_rows(j, e, e, 1)
            pltpu.sync_copy(e, r_sp.at[dst_v.at[j]], add=True)

        _eload(r_lo, e0_v, sem_g0)

        def _rpair(i, _):
            jA = r_lo + 2 * i
            _ewait(e0_v, sem_g0)
            _eload(jA + 1, e1_v, sem_g1)
            _rproc(jA, e0_v)

            _ewait(e1_v, sem_g1)

            @pl.when(jA + 2 < r_lo + r_cnt)
            def _():
                _eload(jA + 2, e0_v, sem_g0)
            _rproc(jA + 1, e1_v)
            return 0
        lax.fori_loop(0, r_cnt // 2, _rpair, 0)

        if r_cnt % 2 == 1:
            jT = r_lo + r_cnt - 1
            _ewait(e0_v, sem_g0)
            _rproc(jT, e0_v)

    def _write_p(pout):
        rows = pl.ds(t * NSL, NSL)
        pltpu.sync_copy(p_sp.at[rows], pout.at[rows])

        @pl.when(t == 15)
        def _():
            tail = pl.ds(9984, 16)
            pltpu.sync_copy(p_sp.at[tail], pout.at[tail])

    def _write_r(rout):
        rows_ = pl.ds(t * NSL, NSL)
        pltpu.sync_copy(r_sp.at[rows_], rout.at[rows_])

        @pl.when(t == 15)
        def _():
            tail_ = pl.ds(9984, 16)
            pltpu.sync_copy(r_sp.at[tail_], rout.at[tail_])

    @pl.when(cid == 0)
    def _():
        _heavy(zq0_hbm)
        _r_phase(0, 62)

    @pl.when(cid == 1)
    def _():
        _heavy(zq2_hbm)
        _r_phase(62, NCH - 62)

    plsc.subcore_barrier()

    @pl.when(cid == 0)
    def _():
        _write_p(p0_out)
        _write_r(r0_out)

    @pl.when(jnp.logical_and(cid == 0, t < 10))
    def _():
        sl1k = pl.ds(t * 1000, 1000)
        pltpu.sync_copy(den_v.at[sl1k], den_out.at[sl1k])

    @pl.when(cid == 1)
    def _():
        _write_p(p2_out)
        _write_r(r1_out)

    _zero_p_sp()
    plsc.subcore_barrier()

    @pl.when(cid == 0)
    def _():
        _heavy(zq1_hbm)

    @pl.when(cid == 1)
    def _():
        _heavy(zq3_hbm)

    plsc.subcore_barrier()

    @pl.when(cid == 0)
    def _():
        _write_p(p1_out)

    @pl.when(cid == 1)
    def _():
        _write_p(p3_out)


@functools.partial(
    pl.kernel,
    mesh=plsc.VectorSubcoreMesh(core_axis_name="c", subcore_axis_name="s"),
    compiler_params=pltpu.CompilerParams(
        needs_layout_passes=False, use_tc_tiling_on_sc=False),
    out_type=[
        jax.ShapeDtypeStruct((N, H), jnp.float32),
        jax.ShapeDtypeStruct((N, H), jnp.float32),
        jax.ShapeDtypeStruct((N, H), jnp.float32),
        jax.ShapeDtypeStruct((N, H), jnp.float32),
        jax.ShapeDtypeStruct((N, ED), jnp.float32),
        jax.ShapeDtypeStruct((N, ED), jnp.float32),
        jax.ShapeDtypeStruct((N,), jnp.float32),
    ],
    scratch_types=[
        pltpu.VMEM((N,), jnp.float32),
        pltpu.VMEM((N,), jnp.float32),
        pltpu.VMEM((NCH, C), jnp.int32),
        pltpu.VMEM((NCH, C), jnp.int32),
        pltpu.VMEM((NCH, C), jnp.float32),
        pltpu.VMEM((N,), jnp.float32),
        pltpu.VMEM((C, H), jnp.float32),
        pltpu.VMEM((C, H), jnp.float32),
        pltpu.VMEM((C, H), jnp.float32),
        pltpu.VMEM((C, ED), jnp.float32),
        pltpu.VMEM((C, ED), jnp.float32),
        pltpu.VMEM((1000,), jnp.float32),
        pltpu.VMEM((16,), jnp.float32),
        pltpu.VMEM((256,), jnp.float32),
        pltpu.VMEM_SHARED((N, H), jnp.float32),
        pltpu.VMEM_SHARED((N, ED), jnp.float32),
        pltpu.VMEM_SHARED((N,), jnp.float32),
        pltpu.VMEM_SHARED((256,), jnp.float32),
        pltpu.SemaphoreType.DMA,
        pltpu.SemaphoreType.DMA,
        pltpu.SemaphoreType.DMA,
        pltpu.SemaphoreType.DMA,
        pltpu.SemaphoreType.DMA,
        pltpu.SemaphoreType.DMA,
    ],
)
def _sc_edge(ei, c, s1, s2, zq0, zq1, zq2, zq3, ef,
             p0, p1, p2, p3, r0, r1, den, *scratch):
    _sc_edge_body(ei, c, s1, s2, zq0, zq1, zq2, zq3, ef,
                  p0, p1, p2, p3, r0, r1, den, *scratch)


def kernel(node_features, edges_features, edge_index, Wn, We, Wa, W2):
    wa_pair = jnp.zeros((D, 128), jnp.float32)
    wa_pair = wa_pair.at[:, 0].set(Wa[:D, 0]).at[:, 1].set(Wa[D + ED:, 0])
    zq0, zq1, zq2, zq3, s_pair = _stage1(node_features, Wn, wa_pair)
    s1 = s_pair[:, 0]
    s2 = s_pair[:, 1]

    ef3 = edges_features.reshape(E // C, C, ED)
    c3 = _stage_c(ef3, We, Wa[D:D + ED]).reshape(16, NCH, C)

    ei4 = edge_index.reshape(2, 16, NCH, C)
    p0, p1, p2, p3, r0, r1, denom = _sc_edge(ei4, c3, s1, s2,
                                             zq0, zq1, zq2, zq3,
                                             edges_features)

    return _stage3((p0, p1, p2, p3), (r0, r1), (zq0, zq1, zq2, zq3),
                   denom[:, None], W2[:D], We, W2[D:])


# speedup vs baseline: 8.5435x; 1.0053x over previous
import functools
import jax
import jax.numpy as jnp
from jax import lax
from jax.experimental import pallas as pl
from jax.experimental.pallas import tpu as pltpu
from jax.experimental.pallas import tpu_sc as plsc

N = 10000
E = 160000
D = 256
ED = 16

NBLK = 10
NROWS = N // NBLK
EROWS = 128
EBLK = (E // 128 + EROWS - 1) // EROWS

ET = E // 16
C = 80
NCH = ET // C
NSL = 624
VSTEPS = ET // 16
H = 64


CROWS = 200


def _stage1_body(x_ref, wn_ref, wa_ref, ef_ref, we_ref, wam_ref,
                 z0_ref, z1_ref, z2_ref, z3_ref, s_ref, c_ref):
    z = jnp.dot(x_ref[...], wn_ref[...], preferred_element_type=jnp.float32)
    z0_ref[...] = z[:, 0:64]
    z1_ref[...] = z[:, 64:128]
    z2_ref[...] = z[:, 128:192]
    z3_ref[...] = z[:, 192:256]
    s_ref[...] = jnp.dot(z, wa_ref[...], preferred_element_type=jnp.float32)
    w_ec = jnp.dot(we_ref[...], wam_ref[...], preferred_element_type=jnp.float32)
    ef = ef_ref[...]
    c_ref[...] = jnp.sum(ef * w_ec[:, 0][None, None, :], axis=-1)


def _stage1(x, Wn, wa_pair, ef3, We, wa_mid):
    zspec = pl.BlockSpec((NROWS, H), lambda i: (i, 0))
    zshape = jax.ShapeDtypeStruct((N, H), jnp.float32)
    return pl.pallas_call(
        _stage1_body,
        grid=(NBLK,),
        in_specs=[
            pl.BlockSpec((NROWS, D), lambda i: (i, 0)),
            pl.BlockSpec((D, D), lambda i: (0, 0)),
            pl.BlockSpec((D, 128), lambda i: (0, 0)),
            pl.BlockSpec((CROWS, C, ED), lambda i: (i, 0, 0)),
            pl.BlockSpec((ED, ED), lambda i: (0, 0)),
            pl.BlockSpec((ED, 1), lambda i: (0, 0)),
        ],
        out_specs=[zspec, zspec, zspec, zspec,
                   pl.BlockSpec((NROWS, 128), lambda i: (i, 0)),
                   pl.BlockSpec((CROWS, C), lambda i: (i, 0))],
        out_shape=[zshape, zshape, zshape, zshape,
                   jax.ShapeDtypeStruct((N, 128), jnp.float32),
                   jax.ShapeDtypeStruct((E // C, C), jnp.float32)],
    )(x, Wn, wa_pair, ef3, We, wa_mid)


def _stage3_body(p0_ref, p1_ref, p2_ref, p3_ref, r0_ref, r1_ref,
                 z0_ref, z1_ref, z2_ref, z3_ref, den_ref,
                 w2a_ref, we_ref, w2b_ref, o_ref):
    w2c = jnp.dot(we_ref[...], w2b_ref[...], preferred_element_type=jnp.float32)
    den = den_ref[...]
    has_msg = den > 0.0
    inv = jnp.where(has_msg, 1.0 / jnp.where(has_msg, den, 1.0), 0.0)
    p = jnp.concatenate(
        [p0_ref[...], p1_ref[...], p2_ref[...], p3_ref[...]], axis=1) * inv
    r = (r0_ref[...] + r1_ref[...]) * inv
    h = jnp.dot(p, w2a_ref[...], preferred_element_type=jnp.float32)
    h = h + jnp.dot(r, w2c, preferred_element_type=jnp.float32)
    z = jnp.concatenate(
        [z0_ref[...], z1_ref[...], z2_ref[...], z3_ref[...]], axis=1)
    o_ref[...] = jnp.where(has_msg, h, z)


def _stage3(ps, rs, zs, denom, W2a, We, W2b):
    zspec = pl.BlockSpec((NROWS, H), lambda i: (i, 0))
    rspec = pl.BlockSpec((NROWS, ED), lambda i: (i, 0))
    return pl.pallas_call(
        _stage3_body,
        grid=(NBLK,),
        in_specs=[
            zspec, zspec, zspec, zspec,
            rspec, rspec,
            zspec, zspec, zspec, zspec,
            pl.BlockSpec((NROWS, 1), lambda i: (i, 0)),
            pl.BlockSpec((D, D), lambda i: (0, 0)),
            pl.BlockSpec((ED, ED), lambda i: (0, 0)),
            pl.BlockSpec((ED, D), lambda i: (0, 0)),
        ],
        out_specs=pl.BlockSpec((NROWS, D), lambda i: (i, 0)),
        out_shape=jax.ShapeDtypeStruct((N, D), jnp.float32),
    )(*ps, *rs, *zs, denom, W2a, We, W2b)


def _f16(v):
    return jnp.full((16,), v, jnp.float32)


def _sc_edge_body(ei_hbm, c_hbm, s1_hbm, s2_hbm,
                  zq0_hbm, zq1_hbm, zq2_hbm, zq3_hbm, ef_hbm,
                  p0_out, p1_out, p2_out, p3_out, r0_out, r1_out, den_out,
                  s1_v, s2_v, src_v, dst_v, ac_v, den_v,
                  g0_v, g1_v, g2_v, e0_v, e1_v,
                  zrow, vec16, m16,
                  p_sp, r_sp, den_sp, gmax_sp,
                  sem_g0, sem_g1, sem_g2, sem_s0, sem_s1, sem_s2):
    t = lax.axis_index("s")
    cid = lax.axis_index("c")
    base = t * ET

    pltpu.sync_copy(s1_hbm, s1_v)
    pltpu.sync_copy(s2_hbm, s2_v)
    pltpu.sync_copy(ei_hbm.at[0, t], src_v)
    pltpu.sync_copy(ei_hbm.at[1, t], dst_v)
    pltpu.sync_copy(c_hbm.at[t], ac_v)

    zero16 = _f16(0.0)

    def _zero_g0(r, _):
        for q in range(H // 16):
            g0_v[r, pl.ds(q * 16, 16)] = zero16
        return 0

    def _zero_e0(r, _):
        e0_v[r, pl.ds(0, 16)] = zero16
        return 0

    def _zr(i, _):
        zrow[pl.ds(i * 16, 16)] = zero16
        return 0
    lax.fori_loop(0, 62, _zr, 0)
    zrow[pl.ds(984, 16)] = zero16

    def _zero_p_sp():
        lax.fori_loop(0, C, _zero_g0, 0)
        for k in range(7):
            pltpu.sync_copy(g0_v.at[pl.ds(0, C)],
                            p_sp.at[pl.ds(t * NSL + k * C, C)])
        pltpu.sync_copy(g0_v.at[pl.ds(0, 64)],
                        p_sp.at[pl.ds(t * NSL + 7 * C, 64)])

        @pl.when(t == 15)
        def _():
            pltpu.sync_copy(g0_v.at[pl.ds(0, 16)], p_sp.at[pl.ds(9984, 16)])

    _zero_p_sp()
    lax.fori_loop(0, C, _zero_e0, 0)
    for k in range(7):
        pltpu.sync_copy(e0_v.at[pl.ds(0, C)],
                        r_sp.at[pl.ds(t * NSL + k * C, C)])
    pltpu.sync_copy(e0_v.at[pl.ds(0, 64)],
                    r_sp.at[pl.ds(t * NSL + 7 * C, 64)])

    @pl.when(t == 15)
    def _():
        pltpu.sync_copy(e0_v.at[pl.ds(0, 16)], r_sp.at[pl.ds(9984, 16)])

    @pl.when(t < 10)
    def _():
        pltpu.sync_copy(zrow, den_sp.at[pl.ds(t * 1000, 1000)])

    def _pass_a(j, m):
        for k in range(C // 16):
            sl = pl.ds(k * 16, 16)
            sv = src_v[j, sl]
            dv = dst_v[j, sl]
            g1 = plsc.load_gather(s1_v, [sv])
            g2 = plsc.load_gather(s2_v, [dv])
            a = g1 + ac_v[j, sl] + g2
            e = jnp.where(a > 0.0, a, 0.01 * a)
            ac_v[j, sl] = e
            m = jnp.maximum(m, e)
        return m
    m = lax.fori_loop(0, NCH, _pass_a, _f16(-1e30))

    vec16[pl.ds(0, 16)] = m
    pltpu.sync_copy(vec16, gmax_sp.at[pl.ds(t * 16, 16)])
    plsc.subcore_barrier()
    pltpu.sync_copy(gmax_sp, m16)

    def _mred(j, mm):
        return jnp.maximum(mm, m16[pl.ds(j * 16, 16)])
    m = lax.fori_loop(0, 16, _mred, _f16(-1e30))
    gv = _f16(jnp.max(m))

    def _pass_b(j, _):
        for k in range(C // 16):
            sl = pl.ds(k * 16, 16)
            ac_v[j, sl] = jnp.exp(ac_v[j, sl] - gv)
        pltpu.async_copy(ac_v.at[j], den_sp.at[dst_v.at[j]], sem_s0,
                         add=True)

        @pl.when(j >= 16)
        def _():
            pltpu.make_async_copy(ac_v.at[j], den_sp.at[dst_v.at[j]],
                                  sem_s0).wait()
        return 0
    lax.fori_loop(0, NCH, _pass_b, 0)

    def _pass_b_drain(i, _):
        pltpu.make_async_copy(ac_v.at[i], den_sp.at[dst_v.at[i]],
                              sem_s0).wait()
        return 0
    lax.fori_loop(0, 16, _pass_b_drain, 0)
    plsc.subcore_barrier()

    @pl.when(jnp.logical_and(cid == 0, t < 10))
    def _():
        sl1k = pl.ds(t * 1000, 1000)
        pltpu.sync_copy(den_sp.at[sl1k], den_v.at[sl1k])

    def _bcast_w(j, r):
        jv = jnp.full((16,), j, jnp.int32)
        rv = jnp.full((16,), r, jnp.int32)
        return plsc.load_gather(ac_v, [jv, rv])

    def _scale_rows(j, gref, sref, nq):
        def _r(r4, _):
            for u in range(4):
                r = r4 * 4 + u
                av = _bcast_w(j, r)
                for q in range(nq):
                    sl = pl.ds(q * 16, 16)
                    sref[r, sl] = gref[r, sl] * av
            return 0
        lax.fori_loop(0, C // 4, _r, 0)

    def _heavy(tbl):
        bufs = (g0_v, g1_v, g2_v)
        gsems = (sem_g0, sem_g1, sem_g2)
        ssems = (sem_s0, sem_s1, sem_s2)

        def _gather(j, g, s):
            pltpu.async_copy(tbl.at[src_v.at[j]], g, s)

        def _gwait(g, s):
            pltpu.make_async_copy(tbl.at[src_v.at[0]], g, s).wait()

        def _swait(j, b, s):
            pltpu.make_async_copy(b, p_sp.at[dst_v.at[j]], s).wait()

        def _step(j, k, first, prefetch):
            g, gs, ss = bufs[k], gsems[k], ssems[k]
            kp = (k + 2) % 3
            _gwait(g, gs)
            if prefetch:
                if first:
                    _gather(j + 2, bufs[kp], gsems[kp])
                else:
                    _swait(j - 1, bufs[kp], ssems[kp])
                    _gather(j + 2, bufs[kp], gsems[kp])
            _scale_rows(j, g, g, H // 16)
            pltpu.async_copy(g, p_sp.at[dst_v.at[j]], ss, add=True)

        _gather(0, g0_v, sem_g0)
        _gather(1, g1_v, sem_g1)
        _step(0, 0, True, True)

        def _triple(i, _):
            j0 = 3 * i
            _step(j0 + 1, 1, False, True)
            _step(j0 + 2, 2, False, True)
            _step(j0 + 3, 0, False, True)
            return 0
        lax.fori_loop(0, (NCH - 5) // 3, _triple, 0)
        _step(NCH - 4, 1, False, True)
        _step(NCH - 3, 2, False, True)
        _step(NCH - 2, 0, False, False)
        _step(NCH - 1, 1, False, False)
        for dj in (NCH - 3, NCH - 2, NCH - 1):
            _swait(dj, bufs[dj % 3], ssems[dj % 3])

    def _r_phase(r_lo, r_cnt):
        def _eload(j, e, s):
            pltpu.async_copy(ef_hbm.at[pl.ds(base + j * C, C)], e, s)

        def _ewait(e, s):
            pltpu.make_async_copy(ef_hbm.at[pl.ds(base, C)], e, s).wait()

        def _rswait(j, e, s):
            pltpu.make_async_copy(e, r_sp.at[dst_v.at[j]], s).wait()

        def _rproc(j, e):
            _scale_rows(j, e, e, 1)
            pltpu.sync_copy(e, r_sp.at[dst_v.at[j]], add=True)

        _eload(r_lo, e0_v, sem_g0)

        def _rpair(i, _):
            jA = r_lo + 2 * i
            _ewait(e0_v, sem_g0)
            _eload(jA + 1, e1_v, sem_g1)
            _rproc(jA, e0_v)

            _ewait(e1_v, sem_g1)

            @pl.when(jA + 2 < r_lo + r_cnt)
            def _():
                _eload(jA + 2, e0_v, sem_g0)
            _rproc(jA + 1, e1_v)
            return 0
        lax.fori_loop(0, r_cnt // 2, _rpair, 0)

        if r_cnt % 2 == 1:
            jT = r_lo + r_cnt - 1
            _ewait(e0_v, sem_g0)
            _rproc(jT, e0_v)

    def _write_p(pout):
        rows = pl.ds(t * NSL, NSL)
        pltpu.sync_copy(p_sp.at[rows], pout.at[rows])

        @pl.when(t == 15)
        def _():
            tail = pl.ds(9984, 16)
            pltpu.sync_copy(p_sp.at[tail], pout.at[tail])

    def _write_r(rout):
        rows_ = pl.ds(t * NSL, NSL)
        pltpu.sync_copy(r_sp.at[rows_], rout.at[rows_])

        @pl.when(t == 15)
        def _():
            tail_ = pl.ds(9984, 16)
            pltpu.sync_copy(r_sp.at[tail_], rout.at[tail_])

    @pl.when(cid == 0)
    def _():
        _heavy(zq0_hbm)
        _r_phase(0, 62)

    @pl.when(cid == 1)
    def _():
        _heavy(zq2_hbm)
        _r_phase(62, NCH - 62)

    plsc.subcore_barrier()

    @pl.when(cid == 0)
    def _():
        _write_p(p0_out)
        _write_r(r0_out)

    @pl.when(jnp.logical_and(cid == 0, t < 10))
    def _():
        sl1k = pl.ds(t * 1000, 1000)
        pltpu.sync_copy(den_v.at[sl1k], den_out.at[sl1k])

    @pl.when(cid == 1)
    def _():
        _write_p(p2_out)
        _write_r(r1_out)

    _zero_p_sp()
    plsc.subcore_barrier()

    @pl.when(cid == 0)
    def _():
        _heavy(zq1_hbm)

    @pl.when(cid == 1)
    def _():
        _heavy(zq3_hbm)

    plsc.subcore_barrier()

    @pl.when(cid == 0)
    def _():
        _write_p(p1_out)

    @pl.when(cid == 1)
    def _():
        _write_p(p3_out)


@functools.partial(
    pl.kernel,
    mesh=plsc.VectorSubcoreMesh(core_axis_name="c", subcore_axis_name="s"),
    compiler_params=pltpu.CompilerParams(
        needs_layout_passes=False, use_tc_tiling_on_sc=False),
    out_type=[
        jax.ShapeDtypeStruct((N, H), jnp.float32),
        jax.ShapeDtypeStruct((N, H), jnp.float32),
        jax.ShapeDtypeStruct((N, H), jnp.float32),
        jax.ShapeDtypeStruct((N, H), jnp.float32),
        jax.ShapeDtypeStruct((N, ED), jnp.float32),
        jax.ShapeDtypeStruct((N, ED), jnp.float32),
        jax.ShapeDtypeStruct((N,), jnp.float32),
    ],
    scratch_types=[
        pltpu.VMEM((N,), jnp.float32),
        pltpu.VMEM((N,), jnp.float32),
        pltpu.VMEM((NCH, C), jnp.int32),
        pltpu.VMEM((NCH, C), jnp.int32),
        pltpu.VMEM((NCH, C), jnp.float32),
        pltpu.VMEM((N,), jnp.float32),
        pltpu.VMEM((C, H), jnp.float32),
        pltpu.VMEM((C, H), jnp.float32),
        pltpu.VMEM((C, H), jnp.float32),
        pltpu.VMEM((C, ED), jnp.float32),
        pltpu.VMEM((C, ED), jnp.float32),
        pltpu.VMEM((1000,), jnp.float32),
        pltpu.VMEM((16,), jnp.float32),
        pltpu.VMEM((256,), jnp.float32),
        pltpu.VMEM_SHARED((N, H), jnp.float32),
        pltpu.VMEM_SHARED((N, ED), jnp.float32),
        pltpu.VMEM_SHARED((N,), jnp.float32),
        pltpu.VMEM_SHARED((256,), jnp.float32),
        pltpu.SemaphoreType.DMA,
        pltpu.SemaphoreType.DMA,
        pltpu.SemaphoreType.DMA,
        pltpu.SemaphoreType.DMA,
        pltpu.SemaphoreType.DMA,
        pltpu.SemaphoreType.DMA,
    ],
)
def _sc_edge(ei, c, s1, s2, zq0, zq1, zq2, zq3, ef,
             p0, p1, p2, p3, r0, r1, den, *scratch):
    _sc_edge_body(ei, c, s1, s2, zq0, zq1, zq2, zq3, ef,
                  p0, p1, p2, p3, r0, r1, den, *scratch)


def kernel(node_features, edges_features, edge_index, Wn, We, Wa, W2):
    wa_pair = jnp.zeros((D, 128), jnp.float32)
    wa_pair = wa_pair.at[:, 0].set(Wa[:D, 0]).at[:, 1].set(Wa[D + ED:, 0])
    ef3 = edges_features.reshape(E // C, C, ED)
    zq0, zq1, zq2, zq3, s_pair, c2 = _stage1(
        node_features, Wn, wa_pair, ef3, We, Wa[D:D + ED])
    s1 = s_pair[:, 0]
    s2 = s_pair[:, 1]
    c3 = c2.reshape(16, NCH, C)

    ei4 = edge_index.reshape(2, 16, NCH, C)
    p0, p1, p2, p3, r0, r1, denom = _sc_edge(ei4, c3, s1, s2,
                                             zq0, zq1, zq2, zq3,
                                             edges_features)

    return _stage3((p0, p1, p2, p3), (r0, r1), (zq0, zq1, zq2, zq3),
                   denom[:, None], W2[:D], We, W2[D:])
